# Initial kernel scaffold; baseline (speedup 1.0000x reference)
import jax
import jax.numpy as jnp
from jax.experimental import pallas as pl


def kernel(xyz_t, neighbor_idx, W, b, gamma, beta):
    raise NotImplementedError("write your pallas kernel here")


# speedup vs baseline: 110.0616x; 110.0616x over previous
import functools

import jax
import jax.numpy as jnp
from jax import lax
from jax.experimental import pallas as pl
from jax.experimental.pallas import tpu as pltpu
from jax.experimental.pallas import tpu_sc as plsc

NEG_SLOPE = 0.01
EPS_BN = 1e-5
LANES = 16


def _sc_gather(xyz_t, idx_flat, K):
    B, _, N = xyz_t.shape
    NK = idx_flat.shape[1]
    info = plsc.get_sparse_core_info()
    NC, NS = info.num_cores, info.num_subcores
    NW = NC * NS
    EW = NK // NW
    assert NK % (NW * LANES) == 0 and EW % 8 == 0
    shift = K.bit_length() - 1
    assert K == 1 << shift

    mesh = plsc.VectorSubcoreMesh(core_axis_name="c", subcore_axis_name="s")

    @functools.partial(
        pl.kernel,
        mesh=mesh,
        compiler_params=pltpu.CompilerParams(needs_layout_passes=False),
        out_type=jax.ShapeDtypeStruct((B * 6 * NK,), jnp.float32),
        scratch_types=[
            pltpu.VMEM((N,), jnp.float32),
            pltpu.VMEM((N,), jnp.float32),
            pltpu.VMEM((N,), jnp.float32),
            pltpu.VMEM((EW,), jnp.int32),
            pltpu.VMEM((EW,), jnp.float32),
            pltpu.VMEM((EW,), jnp.float32),
            pltpu.VMEM((EW,), jnp.float32),
            pltpu.VMEM((EW,), jnp.float32),
            pltpu.VMEM((EW,), jnp.float32),
            pltpu.VMEM((EW,), jnp.float32),
        ],
    )
    def k(xyz_hbm, idx_hbm, out_hbm, x_v, y_v, z_v, idx_v,
          px_v, py_v, pz_v, qx_v, qy_v, qz_v):
        wid = lax.axis_index("s") * NC + lax.axis_index("c")
        base = wid * EW
        lane = lax.iota(jnp.int32, LANES)
        for b in range(B):
            pltpu.sync_copy(idx_hbm.at[pl.ds(b * NK + base, EW)], idx_v)
            pltpu.sync_copy(xyz_hbm.at[pl.ds((b * 3 + 0) * N, N)], x_v)
            pltpu.sync_copy(xyz_hbm.at[pl.ds((b * 3 + 1) * N, N)], y_v)
            pltpu.sync_copy(xyz_hbm.at[pl.ds((b * 3 + 2) * N, N)], z_v)

            def body(i, carry):
                off = i * LANES
                iv = idx_v[pl.ds(off, LANES)]
                pv = lax.shift_right_logical(lane + (base + off), shift)
                px_v[pl.ds(off, LANES)] = plsc.load_gather(x_v, [pv])
                py_v[pl.ds(off, LANES)] = plsc.load_gather(y_v, [pv])
                pz_v[pl.ds(off, LANES)] = plsc.load_gather(z_v, [pv])
                qx_v[pl.ds(off, LANES)] = plsc.load_gather(x_v, [iv])
                qy_v[pl.ds(off, LANES)] = plsc.load_gather(y_v, [iv])
                qz_v[pl.ds(off, LANES)] = plsc.load_gather(z_v, [iv])
                return carry

            lax.fori_loop(0, EW // LANES, body, 0)
            pltpu.sync_copy(px_v, out_hbm.at[pl.ds((b * 6 + 0) * NK + base, EW)])
            pltpu.sync_copy(py_v, out_hbm.at[pl.ds((b * 6 + 1) * NK + base, EW)])
            pltpu.sync_copy(pz_v, out_hbm.at[pl.ds((b * 6 + 2) * NK + base, EW)])
            pltpu.sync_copy(qx_v, out_hbm.at[pl.ds((b * 6 + 3) * NK + base, EW)])
            pltpu.sync_copy(qy_v, out_hbm.at[pl.ds((b * 6 + 4) * NK + base, EW)])
            pltpu.sync_copy(qz_v, out_hbm.at[pl.ds((b * 6 + 5) * NK + base, EW)])

    return k(xyz_t.reshape(B * 3 * N), idx_flat.reshape(B * NK)).reshape(2 * B, 3, NK)


def _z8(p_ref, q_ref, BLK):
    P = p_ref[0]
    Q = q_ref[0]
    rel = P - Q
    s = jnp.sum(rel * rel, axis=0, keepdims=True)
    dist = jnp.sqrt(s)
    ones = jnp.ones((1, BLK), jnp.float32)
    return jnp.concatenate([dist, P, Q, ones], axis=0)


def _tc_stats(pq, W8, g2, be2, B, BLK, T):
    NK = pq.shape[2]
    inv_cnt = 1.0 / float(B * NK)

    def body(p_ref, q_ref, w_ref, g_ref, be_ref, wp_ref, m_ref):
        b, t = pl.program_id(0), pl.program_id(1)
        Z = _z8(p_ref, q_ref, BLK)
        m = lax.dot_general(Z, Z, (((1,), (1,)), ((), ())),
                            preferred_element_type=jnp.float32)

        @pl.when((b == 0) & (t == 0))
        def _():
            m_ref[...] = jnp.zeros_like(m_ref)

        m_ref[...] += m

        @pl.when((b == B - 1) & (t == T - 1))
        def _():
            w8 = w_ref[...]
            wm = jnp.dot(w8, m_ref[...], preferred_element_type=jnp.float32,
                         precision=lax.Precision.HIGHEST)
            mean = wm[:, 7:8] * inv_cnt
            ey2 = jnp.sum(wm * w8, axis=1, keepdims=True) * inv_cnt
            var = ey2 - mean * mean
            scale = g_ref[...] / jnp.sqrt(var + EPS_BN)
            shift = be_ref[...] - scale * mean
            col = lax.broadcasted_iota(jnp.int32, (16, 8), 1)
            wp_ref[...] = scale * w8 + jnp.where(col == 7, shift, 0.0)

    return pl.pallas_call(
        body,
        grid=(B, T),
        in_specs=[
            pl.BlockSpec((1, 3, BLK), lambda b, t: (2 * b, 0, t)),
            pl.BlockSpec((1, 3, BLK), lambda b, t: (2 * b + 1, 0, t)),
            pl.BlockSpec((16, 8), lambda b, t: (0, 0)),
            pl.BlockSpec((16, 1), lambda b, t: (0, 0)),
            pl.BlockSpec((16, 1), lambda b, t: (0, 0)),
        ],
        out_specs=pl.BlockSpec((16, 8), lambda b, t: (0, 0)),
        out_shape=jax.ShapeDtypeStruct((16, 8), jnp.float32),
        scratch_shapes=[pltpu.VMEM((8, 8), jnp.float32)],
    )(pq, pq, W8, g2, be2)


def _tc_emit(pq, W8p, B, BLK, T):
    NK = pq.shape[2]

    def body(p_ref, q_ref, w_ref, o_ref):
        Z = _z8(p_ref, q_ref, BLK)
        yn = jnp.dot(w_ref[...], Z, preferred_element_type=jnp.float32,
                     precision=lax.Precision.HIGHEST)
        o_ref[0] = jnp.maximum(yn, NEG_SLOPE * yn)

    return pl.pallas_call(
        body,
        grid=(B, T),
        in_specs=[
            pl.BlockSpec((1, 3, BLK), lambda b, t: (2 * b, 0, t)),
            pl.BlockSpec((1, 3, BLK), lambda b, t: (2 * b + 1, 0, t)),
            pl.BlockSpec((16, 8), lambda b, t: (0, 0)),
        ],
        out_specs=pl.BlockSpec((1, 16, BLK), lambda b, t: (b, 0, t)),
        out_shape=jax.ShapeDtypeStruct((B, 16, NK), jnp.float32),
    )(pq, pq, W8p)


def kernel(xyz_t, neighbor_idx, W, b, gamma, beta):
    B, _, N = xyz_t.shape
    K = neighbor_idx.shape[-1]
    NK = N * K
    idx_flat = neighbor_idx.reshape(B, NK).astype(jnp.int32)

    pq = _sc_gather(xyz_t, idx_flat, K)

    BLK = 6400
    assert NK % BLK == 0
    T = NK // BLK

    W8 = jnp.concatenate(
        [W[:, 0:1], W[:, 1:4] + W[:, 4:7], W[:, 7:10] - W[:, 1:4],
         b.reshape(16, 1)], axis=1)

    g2 = gamma.reshape(16, 1)
    be2 = beta.reshape(16, 1)
    W8p = _tc_stats(pq, W8, g2, be2, B, BLK, T)
    return _tc_emit(pq, W8p, B, BLK, T)


# speedup vs baseline: 187.5868x; 1.7044x over previous
import functools

import jax
import jax.numpy as jnp
from jax import lax
from jax.experimental import pallas as pl
from jax.experimental.pallas import tpu as pltpu
from jax.experimental.pallas import tpu_sc as plsc

NEG_SLOPE = 0.01
EPS_BN = 1e-5
LANES = 16


def _sc_gather(xyz_t, idx_flat, K):
    B, _, N = xyz_t.shape
    NK = idx_flat.shape[1]
    info = plsc.get_sparse_core_info()
    NC, NS = info.num_cores, info.num_subcores
    NW = NC * NS
    EW = NK // NW
    assert NK % (NW * LANES) == 0 and EW % 8 == 0
    shift = K.bit_length() - 1
    assert K == 1 << shift

    mesh = plsc.VectorSubcoreMesh(core_axis_name="c", subcore_axis_name="s")

    @functools.partial(
        pl.kernel,
        mesh=mesh,
        compiler_params=pltpu.CompilerParams(needs_layout_passes=False),
        out_type=jax.ShapeDtypeStruct((B * 6 * NK,), jnp.float32),
        scratch_types=[
            pltpu.VMEM((N,), jnp.float32),
            pltpu.VMEM((N,), jnp.float32),
            pltpu.VMEM((N,), jnp.float32),
            pltpu.VMEM((EW,), jnp.int32),
            pltpu.VMEM((EW,), jnp.float32),
            pltpu.VMEM((EW,), jnp.float32),
            pltpu.VMEM((EW,), jnp.float32),
            pltpu.VMEM((EW,), jnp.float32),
            pltpu.VMEM((EW,), jnp.float32),
            pltpu.VMEM((EW,), jnp.float32),
        ],
    )
    def k(xyz_hbm, idx_hbm, out_hbm, x_v, y_v, z_v, idx_v,
          px_v, py_v, pz_v, qx_v, qy_v, qz_v):
        wid = lax.axis_index("s") * NC + lax.axis_index("c")
        base = wid * EW
        lane = lax.iota(jnp.int32, LANES)
        for b in range(B):
            pltpu.sync_copy(idx_hbm.at[pl.ds(b * NK + base, EW)], idx_v)
            pltpu.sync_copy(xyz_hbm.at[pl.ds((b * 3 + 0) * N, N)], x_v)
            pltpu.sync_copy(xyz_hbm.at[pl.ds((b * 3 + 1) * N, N)], y_v)
            pltpu.sync_copy(xyz_hbm.at[pl.ds((b * 3 + 2) * N, N)], z_v)

            @plsc.parallel_loop(0, EW, LANES, unroll=8)
            def body(off):
                iv = idx_v[pl.ds(off, LANES)]
                pv = lax.shift_right_logical(lane + (base + off), shift)
                px_v[pl.ds(off, LANES)] = plsc.load_gather(x_v, [pv])
                py_v[pl.ds(off, LANES)] = plsc.load_gather(y_v, [pv])
                pz_v[pl.ds(off, LANES)] = plsc.load_gather(z_v, [pv])
                qx_v[pl.ds(off, LANES)] = plsc.load_gather(x_v, [iv])
                qy_v[pl.ds(off, LANES)] = plsc.load_gather(y_v, [iv])
                qz_v[pl.ds(off, LANES)] = plsc.load_gather(z_v, [iv])
            pltpu.sync_copy(px_v, out_hbm.at[pl.ds((b * 6 + 0) * NK + base, EW)])
            pltpu.sync_copy(py_v, out_hbm.at[pl.ds((b * 6 + 1) * NK + base, EW)])
            pltpu.sync_copy(pz_v, out_hbm.at[pl.ds((b * 6 + 2) * NK + base, EW)])
            pltpu.sync_copy(qx_v, out_hbm.at[pl.ds((b * 6 + 3) * NK + base, EW)])
            pltpu.sync_copy(qy_v, out_hbm.at[pl.ds((b * 6 + 4) * NK + base, EW)])
            pltpu.sync_copy(qz_v, out_hbm.at[pl.ds((b * 6 + 5) * NK + base, EW)])

    return k(xyz_t.reshape(B * 3 * N), idx_flat.reshape(B * NK)).reshape(2 * B, 3, NK)


def _z8(p_ref, q_ref, BLK):
    P = p_ref[0]
    Q = q_ref[0]
    rel = P - Q
    s = jnp.sum(rel * rel, axis=0, keepdims=True)
    dist = jnp.sqrt(s)
    ones = jnp.ones((1, BLK), jnp.float32)
    return jnp.concatenate([dist, P, Q, ones], axis=0)


def _tc_stats(pq, W8, g2, be2, B, BLK, T):
    NK = pq.shape[2]
    inv_cnt = 1.0 / float(B * NK)

    def body(p_ref, q_ref, w_ref, g_ref, be_ref, wp_ref, m_ref):
        b, t = pl.program_id(0), pl.program_id(1)
        Z = _z8(p_ref, q_ref, BLK)
        m = lax.dot_general(Z, Z, (((1,), (1,)), ((), ())),
                            preferred_element_type=jnp.float32)

        @pl.when((b == 0) & (t == 0))
        def _():
            m_ref[...] = jnp.zeros_like(m_ref)

        m_ref[...] += m

        @pl.when((b == B - 1) & (t == T - 1))
        def _():
            w8 = w_ref[...]
            wm = jnp.dot(w8, m_ref[...], preferred_element_type=jnp.float32,
                         precision=lax.Precision.HIGHEST)
            mean = wm[:, 7:8] * inv_cnt
            ey2 = jnp.sum(wm * w8, axis=1, keepdims=True) * inv_cnt
            var = ey2 - mean * mean
            scale = g_ref[...] / jnp.sqrt(var + EPS_BN)
            shift = be_ref[...] - scale * mean
            col = lax.broadcasted_iota(jnp.int32, (16, 8), 1)
            wp_ref[...] = scale * w8 + jnp.where(col == 7, shift, 0.0)

    return pl.pallas_call(
        body,
        grid=(B, T),
        in_specs=[
            pl.BlockSpec((1, 3, BLK), lambda b, t: (2 * b, 0, t)),
            pl.BlockSpec((1, 3, BLK), lambda b, t: (2 * b + 1, 0, t)),
            pl.BlockSpec((16, 8), lambda b, t: (0, 0)),
            pl.BlockSpec((16, 1), lambda b, t: (0, 0)),
            pl.BlockSpec((16, 1), lambda b, t: (0, 0)),
        ],
        out_specs=pl.BlockSpec((16, 8), lambda b, t: (0, 0)),
        out_shape=jax.ShapeDtypeStruct((16, 8), jnp.float32),
        scratch_shapes=[pltpu.VMEM((8, 8), jnp.float32)],
    )(pq, pq, W8, g2, be2)


def _tc_emit(pq, W8p, B, BLK, T):
    NK = pq.shape[2]

    def body(p_ref, q_ref, w_ref, o_ref):
        Z = _z8(p_ref, q_ref, BLK)
        yn = jnp.dot(w_ref[...], Z, preferred_element_type=jnp.float32)
        o_ref[0] = jnp.maximum(yn, NEG_SLOPE * yn)

    return pl.pallas_call(
        body,
        grid=(B, T),
        in_specs=[
            pl.BlockSpec((1, 3, BLK), lambda b, t: (2 * b, 0, t)),
            pl.BlockSpec((1, 3, BLK), lambda b, t: (2 * b + 1, 0, t)),
            pl.BlockSpec((16, 8), lambda b, t: (0, 0)),
        ],
        out_specs=pl.BlockSpec((1, 16, BLK), lambda b, t: (b, 0, t)),
        out_shape=jax.ShapeDtypeStruct((B, 16, NK), jnp.float32),
    )(pq, pq, W8p)


def kernel(xyz_t, neighbor_idx, W, b, gamma, beta):
    B, _, N = xyz_t.shape
    K = neighbor_idx.shape[-1]
    NK = N * K
    idx_flat = neighbor_idx.reshape(B, NK).astype(jnp.int32)

    pq = _sc_gather(xyz_t, idx_flat, K)

    BLK = 16000
    assert NK % BLK == 0
    T = NK // BLK

    W8 = jnp.concatenate(
        [W[:, 0:1], W[:, 1:4] + W[:, 4:7], W[:, 7:10] - W[:, 1:4],
         b.reshape(16, 1)], axis=1)

    g2 = gamma.reshape(16, 1)
    be2 = beta.reshape(16, 1)
    W8p = _tc_stats(pq, W8, g2, be2, B, BLK, T)
    return _tc_emit(pq, W8p, B, BLK, T)


# speedup vs baseline: 223.0096x; 1.1888x over previous
import functools

import jax
import jax.numpy as jnp
from jax import lax
from jax.experimental import pallas as pl
from jax.experimental.pallas import tpu as pltpu
from jax.experimental.pallas import tpu_sc as plsc

NEG_SLOPE = 0.01
EPS_BN = 1e-5
LANES = 16


def _sc_gather(xyz_t, idx_flat, K):
    B, _, N = xyz_t.shape
    NK = idx_flat.shape[1]
    info = plsc.get_sparse_core_info()
    NC, NS = info.num_cores, info.num_subcores
    NW = NC * NS
    EW = NK // NW
    assert NK % (NW * LANES) == 0 and EW % 8 == 0
    shift = K.bit_length() - 1
    assert K == 1 << shift

    mesh = plsc.VectorSubcoreMesh(core_axis_name="c", subcore_axis_name="s")

    @functools.partial(
        pl.kernel,
        mesh=mesh,
        compiler_params=pltpu.CompilerParams(needs_layout_passes=False),
        out_type=jax.ShapeDtypeStruct((B * 6 * NK,), jnp.float32),
        scratch_types=[
            pltpu.VMEM((N,), jnp.float32),
            pltpu.VMEM((N,), jnp.float32),
            pltpu.VMEM((N,), jnp.float32),
            pltpu.VMEM((EW,), jnp.int32),
            pltpu.VMEM((EW,), jnp.float32),
            pltpu.VMEM((EW,), jnp.float32),
            pltpu.VMEM((EW,), jnp.float32),
            pltpu.VMEM((EW,), jnp.float32),
            pltpu.VMEM((EW,), jnp.float32),
            pltpu.VMEM((EW,), jnp.float32),
        ],
    )
    def k(xyz_hbm, idx_hbm, out_hbm, x_v, y_v, z_v, idx_v,
          px_v, py_v, pz_v, qx_v, qy_v, qz_v):
        wid = lax.axis_index("s") * NC + lax.axis_index("c")
        base = wid * EW
        lane = lax.iota(jnp.int32, LANES)
        for b in range(B):
            pltpu.sync_copy(idx_hbm.at[pl.ds(b * NK + base, EW)], idx_v)
            pltpu.sync_copy(xyz_hbm.at[pl.ds((b * 3 + 0) * N, N)], x_v)
            pltpu.sync_copy(xyz_hbm.at[pl.ds((b * 3 + 1) * N, N)], y_v)
            pltpu.sync_copy(xyz_hbm.at[pl.ds((b * 3 + 2) * N, N)], z_v)

            @plsc.parallel_loop(0, EW, LANES, unroll=8)
            def body(off):
                iv = idx_v[pl.ds(off, LANES)]
                pv = lax.shift_right_logical(lane + (base + off), shift)
                px_v[pl.ds(off, LANES)] = plsc.load_gather(x_v, [pv])
                py_v[pl.ds(off, LANES)] = plsc.load_gather(y_v, [pv])
                pz_v[pl.ds(off, LANES)] = plsc.load_gather(z_v, [pv])
                qx_v[pl.ds(off, LANES)] = plsc.load_gather(x_v, [iv])
                qy_v[pl.ds(off, LANES)] = plsc.load_gather(y_v, [iv])
                qz_v[pl.ds(off, LANES)] = plsc.load_gather(z_v, [iv])
            pltpu.sync_copy(px_v, out_hbm.at[pl.ds((b * 6 + 0) * NK + base, EW)])
            pltpu.sync_copy(py_v, out_hbm.at[pl.ds((b * 6 + 1) * NK + base, EW)])
            pltpu.sync_copy(pz_v, out_hbm.at[pl.ds((b * 6 + 2) * NK + base, EW)])
            pltpu.sync_copy(qx_v, out_hbm.at[pl.ds((b * 6 + 3) * NK + base, EW)])
            pltpu.sync_copy(qy_v, out_hbm.at[pl.ds((b * 6 + 4) * NK + base, EW)])
            pltpu.sync_copy(qz_v, out_hbm.at[pl.ds((b * 6 + 5) * NK + base, EW)])

    return k(xyz_t.reshape(B * 3 * N), idx_flat.reshape(B * NK)).reshape(2 * B, 3, NK)


def _z8(p_ref, q_ref, BLK):
    P = p_ref[0]
    Q = q_ref[0]
    rel = P - Q
    s = jnp.sum(rel * rel, axis=0, keepdims=True)
    dist = s * lax.rsqrt(s + 1e-30)
    ones = jnp.ones((1, BLK), jnp.float32)
    return jnp.concatenate([P, Q, dist, ones], axis=0)


def _tc_stats(pq, W8, g2, be2, B, BLK, T):
    NK = pq.shape[2]
    inv_cnt = 1.0 / float(B * NK)

    def body(p_ref, q_ref, w_ref, g_ref, be_ref, wp_ref, m_ref):
        b, t = pl.program_id(0), pl.program_id(1)
        Z = _z8(p_ref, q_ref, BLK)
        m = lax.dot_general(Z, Z, (((1,), (1,)), ((), ())),
                            preferred_element_type=jnp.float32)

        @pl.when((b == 0) & (t == 0))
        def _():
            m_ref[...] = jnp.zeros_like(m_ref)

        m_ref[...] += m

        @pl.when((b == B - 1) & (t == T - 1))
        def _():
            w8 = w_ref[...]
            wm = jnp.dot(w8, m_ref[...], preferred_element_type=jnp.float32,
                         precision=lax.Precision.HIGHEST)
            mean = wm[:, 7:8] * inv_cnt
            ey2 = jnp.sum(wm * w8, axis=1, keepdims=True) * inv_cnt
            var = ey2 - mean * mean
            scale = g_ref[...] / jnp.sqrt(var + EPS_BN)
            shift = be_ref[...] - scale * mean
            col = lax.broadcasted_iota(jnp.int32, (16, 8), 1)
            wp_ref[...] = scale * w8 + jnp.where(col == 7, shift, 0.0)

    return pl.pallas_call(
        body,
        grid=(B, T),
        in_specs=[
            pl.BlockSpec((1, 3, BLK), lambda b, t: (2 * b, 0, t)),
            pl.BlockSpec((1, 3, BLK), lambda b, t: (2 * b + 1, 0, t)),
            pl.BlockSpec((16, 8), lambda b, t: (0, 0)),
            pl.BlockSpec((16, 1), lambda b, t: (0, 0)),
            pl.BlockSpec((16, 1), lambda b, t: (0, 0)),
        ],
        out_specs=pl.BlockSpec((16, 8), lambda b, t: (0, 0)),
        out_shape=jax.ShapeDtypeStruct((16, 8), jnp.float32),
        scratch_shapes=[pltpu.VMEM((8, 8), jnp.float32)],
    )(pq, pq, W8, g2, be2)


def _tc_emit(pq, W8p, B, BLK, T):
    NK = pq.shape[2]

    def body(p_ref, q_ref, w_ref, o_ref):
        Z = _z8(p_ref, q_ref, BLK)
        yn = jnp.dot(w_ref[...], Z, preferred_element_type=jnp.float32)
        o_ref[0] = jnp.maximum(yn, NEG_SLOPE * yn)

    return pl.pallas_call(
        body,
        grid=(B, T),
        in_specs=[
            pl.BlockSpec((1, 3, BLK), lambda b, t: (2 * b, 0, t)),
            pl.BlockSpec((1, 3, BLK), lambda b, t: (2 * b + 1, 0, t)),
            pl.BlockSpec((16, 8), lambda b, t: (0, 0)),
        ],
        out_specs=pl.BlockSpec((1, 16, BLK), lambda b, t: (b, 0, t)),
        out_shape=jax.ShapeDtypeStruct((B, 16, NK), jnp.float32),
    )(pq, pq, W8p)


def kernel(xyz_t, neighbor_idx, W, b, gamma, beta):
    B, _, N = xyz_t.shape
    K = neighbor_idx.shape[-1]
    NK = N * K
    idx_flat = neighbor_idx.reshape(B, NK).astype(jnp.int32)

    pq = _sc_gather(xyz_t, idx_flat, K)

    BLK = 32000
    assert NK % BLK == 0
    T = NK // BLK

    W8 = jnp.concatenate(
        [W[:, 1:4] + W[:, 4:7], W[:, 7:10] - W[:, 1:4], W[:, 0:1],
         b.reshape(16, 1)], axis=1)

    g2 = gamma.reshape(16, 1)
    be2 = beta.reshape(16, 1)
    W8p = _tc_stats(pq, W8, g2, be2, B, BLK, T)
    return _tc_emit(pq, W8p, B, BLK, T)


# speedup vs baseline: 244.1428x; 1.0948x over previous
import functools

import jax
import jax.numpy as jnp
from jax import lax
from jax.experimental import pallas as pl
from jax.experimental.pallas import tpu as pltpu
from jax.experimental.pallas import tpu_sc as plsc

NEG_SLOPE = 0.01
EPS_BN = 1e-5
LANES = 16


def _sc_gather(xyz_t, idx_flat, K):
    B, _, N = xyz_t.shape
    NK = idx_flat.shape[1]
    info = plsc.get_sparse_core_info()
    NC, NS = info.num_cores, info.num_subcores
    NW = NC * NS
    EW = NK // NW
    assert NK % (NW * LANES) == 0 and EW % 8 == 0
    shift = K.bit_length() - 1
    assert K == 1 << shift

    mesh = plsc.VectorSubcoreMesh(core_axis_name="c", subcore_axis_name="s")

    @functools.partial(
        pl.kernel,
        mesh=mesh,
        compiler_params=pltpu.CompilerParams(needs_layout_passes=False),
        out_type=jax.ShapeDtypeStruct((B * 6 * NK,), jnp.float32),
        scratch_types=[
            pltpu.VMEM((N,), jnp.float32),
            pltpu.VMEM((N,), jnp.float32),
            pltpu.VMEM((N,), jnp.float32),
            pltpu.VMEM((EW,), jnp.int32),
            pltpu.VMEM((EW,), jnp.float32),
            pltpu.VMEM((EW,), jnp.float32),
            pltpu.VMEM((EW,), jnp.float32),
            pltpu.VMEM((EW,), jnp.float32),
            pltpu.VMEM((EW,), jnp.float32),
            pltpu.VMEM((EW,), jnp.float32),
        ],
    )
    def k(xyz_hbm, idx_hbm, out_hbm, x_v, y_v, z_v, idx_v,
          px_v, py_v, pz_v, qx_v, qy_v, qz_v):
        wid = lax.axis_index("s") * NC + lax.axis_index("c")
        base = wid * EW
        lane = lax.iota(jnp.int32, LANES)
        for b in range(B):
            pltpu.sync_copy(idx_hbm.at[pl.ds(b * NK + base, EW)], idx_v)
            pltpu.sync_copy(xyz_hbm.at[pl.ds((b * 3 + 0) * N, N)], x_v)
            pltpu.sync_copy(xyz_hbm.at[pl.ds((b * 3 + 1) * N, N)], y_v)
            pltpu.sync_copy(xyz_hbm.at[pl.ds((b * 3 + 2) * N, N)], z_v)

            @plsc.parallel_loop(0, EW, LANES, unroll=8)
            def body(off):
                iv = idx_v[pl.ds(off, LANES)]
                pv = lax.shift_right_logical(lane + (base + off), shift)
                px_v[pl.ds(off, LANES)] = plsc.load_gather(x_v, [pv])
                py_v[pl.ds(off, LANES)] = plsc.load_gather(y_v, [pv])
                pz_v[pl.ds(off, LANES)] = plsc.load_gather(z_v, [pv])
                qx_v[pl.ds(off, LANES)] = plsc.load_gather(x_v, [iv])
                qy_v[pl.ds(off, LANES)] = plsc.load_gather(y_v, [iv])
                qz_v[pl.ds(off, LANES)] = plsc.load_gather(z_v, [iv])
            pltpu.sync_copy(px_v, out_hbm.at[pl.ds((b * 6 + 0) * NK + base, EW)])
            pltpu.sync_copy(py_v, out_hbm.at[pl.ds((b * 6 + 1) * NK + base, EW)])
            pltpu.sync_copy(pz_v, out_hbm.at[pl.ds((b * 6 + 2) * NK + base, EW)])
            pltpu.sync_copy(qx_v, out_hbm.at[pl.ds((b * 6 + 3) * NK + base, EW)])
            pltpu.sync_copy(qy_v, out_hbm.at[pl.ds((b * 6 + 4) * NK + base, EW)])
            pltpu.sync_copy(qz_v, out_hbm.at[pl.ds((b * 6 + 5) * NK + base, EW)])

    return k(xyz_t.reshape(B * 3 * N), idx_flat.reshape(B * NK)).reshape(2 * B, 3, NK)


def _z8(p_ref, q_ref, BLK):
    P = p_ref[0]
    Q = q_ref[0]
    rel = P - Q
    s = jnp.sum(rel * rel, axis=0, keepdims=True)
    dist = s * lax.rsqrt(s + 1e-30)
    ones = jnp.ones((1, BLK), jnp.float32)
    return jnp.concatenate([P, Q, dist, ones], axis=0)


def _tc_stats(pq, W8, g2, be2, B, BLK, T):
    NK = pq.shape[2]
    inv_cnt = 1.0 / float(B * NK)

    def body(p_ref, q_ref, w_ref, g_ref, be_ref, wp_ref, m_ref):
        b, t = pl.program_id(0), pl.program_id(1)
        Z = _z8(p_ref, q_ref, BLK)
        m = lax.dot_general(Z, Z, (((1,), (1,)), ((), ())),
                            preferred_element_type=jnp.float32)

        @pl.when((b == 0) & (t == 0))
        def _():
            m_ref[...] = jnp.zeros_like(m_ref)

        m_ref[...] += m

        @pl.when((b == B - 1) & (t == T - 1))
        def _():
            w8 = w_ref[...]
            wm = jnp.dot(w8, m_ref[...], preferred_element_type=jnp.float32,
                         precision=lax.Precision.HIGHEST)
            mean = wm[:, 7:8] * inv_cnt
            ey2 = jnp.sum(wm * w8, axis=1, keepdims=True) * inv_cnt
            var = ey2 - mean * mean
            scale = g_ref[...] / jnp.sqrt(var + EPS_BN)
            shift = be_ref[...] - scale * mean
            col = lax.broadcasted_iota(jnp.int32, (16, 8), 1)
            wp_ref[...] = scale * w8 + jnp.where(col == 7, shift, 0.0)

    return pl.pallas_call(
        body,
        grid=(B, T),
        in_specs=[
            pl.BlockSpec((1, 3, BLK), lambda b, t: (2 * b, 0, t)),
            pl.BlockSpec((1, 3, BLK), lambda b, t: (2 * b + 1, 0, t)),
            pl.BlockSpec((16, 8), lambda b, t: (0, 0)),
            pl.BlockSpec((16, 1), lambda b, t: (0, 0)),
            pl.BlockSpec((16, 1), lambda b, t: (0, 0)),
        ],
        out_specs=pl.BlockSpec((16, 8), lambda b, t: (0, 0)),
        out_shape=jax.ShapeDtypeStruct((16, 8), jnp.float32),
        scratch_shapes=[pltpu.VMEM((8, 8), jnp.float32)],
    )(pq, pq, W8, g2, be2)


def _tc_emit(pq, W8p, B, BLK, T):
    NK = pq.shape[2]

    def body(p_ref, q_ref, w_ref, o_ref):
        Z = _z8(p_ref, q_ref, BLK)
        yn = jnp.dot(w_ref[...], Z, preferred_element_type=jnp.float32)
        o_ref[0] = jnp.maximum(yn, NEG_SLOPE * yn)

    return pl.pallas_call(
        body,
        grid=(B, T),
        in_specs=[
            pl.BlockSpec((1, 3, BLK), lambda b, t: (2 * b, 0, t)),
            pl.BlockSpec((1, 3, BLK), lambda b, t: (2 * b + 1, 0, t)),
            pl.BlockSpec((16, 8), lambda b, t: (0, 0)),
        ],
        out_specs=pl.BlockSpec((1, 16, BLK), lambda b, t: (b, 0, t)),
        out_shape=jax.ShapeDtypeStruct((B, 16, NK), jnp.float32),
    )(pq, pq, W8p)


def kernel(xyz_t, neighbor_idx, W, b, gamma, beta):
    B, _, N = xyz_t.shape
    K = neighbor_idx.shape[-1]
    NK = N * K
    idx_flat = neighbor_idx.reshape(B, NK).astype(jnp.int32)

    pq = _sc_gather(xyz_t, idx_flat, K)

    BLK = 64000
    assert NK % BLK == 0
    T = NK // BLK

    W8 = jnp.concatenate(
        [W[:, 1:4] + W[:, 4:7], W[:, 7:10] - W[:, 1:4], W[:, 0:1],
         b.reshape(16, 1)], axis=1)

    g2 = gamma.reshape(16, 1)
    be2 = beta.reshape(16, 1)
    W8p = _tc_stats(pq, W8, g2, be2, B, BLK, T)
    return _tc_emit(pq, W8p, B, BLK, T)


# speedup vs baseline: 260.9143x; 1.0687x over previous
import functools

import jax
import jax.numpy as jnp
from jax import lax
from jax.experimental import pallas as pl
from jax.experimental.pallas import tpu as pltpu
from jax.experimental.pallas import tpu_sc as plsc

NEG_SLOPE = 0.01
EPS_BN = 1e-5
LANES = 16


def _sc_gather(xyz_t, idx_flat, K):
    B, _, N = xyz_t.shape
    NK = idx_flat.shape[1]
    info = plsc.get_sparse_core_info()
    NC, NS = info.num_cores, info.num_subcores
    NW = NC * NS
    EW = -(-NK // (NW * 256)) * 256
    NKp = EW * NW
    assert EW % (2 * LANES) == 0
    shift = K.bit_length() - 1
    assert K == 1 << shift
    idx_pad = jnp.pad(idx_flat, ((0, 0), (0, NKp - NK)))

    mesh = plsc.VectorSubcoreMesh(core_axis_name="c", subcore_axis_name="s")

    @functools.partial(
        pl.kernel,
        mesh=mesh,
        compiler_params=pltpu.CompilerParams(needs_layout_passes=False),
        out_type=jax.ShapeDtypeStruct((B * 6 * NKp,), jnp.bfloat16),
        scratch_types=[
            pltpu.VMEM((N,), jnp.float32),
            pltpu.VMEM((N,), jnp.float32),
            pltpu.VMEM((N,), jnp.float32),
            pltpu.VMEM((EW,), jnp.int32),
            pltpu.VMEM((EW,), jnp.bfloat16),
            pltpu.VMEM((EW,), jnp.bfloat16),
            pltpu.VMEM((EW,), jnp.bfloat16),
            pltpu.VMEM((EW,), jnp.bfloat16),
            pltpu.VMEM((EW,), jnp.bfloat16),
            pltpu.VMEM((EW,), jnp.bfloat16),
        ],
    )
    def k(xyz_hbm, idx_hbm, out_hbm, x_v, y_v, z_v, idx_v,
          px_v, py_v, pz_v, qx_v, qy_v, qz_v):
        wid = lax.axis_index("s") * NC + lax.axis_index("c")
        base = wid * EW
        lane = lax.iota(jnp.int32, LANES)
        for b in range(B):
            pltpu.sync_copy(idx_hbm.at[pl.ds(b * NKp + base, EW)], idx_v)
            pltpu.sync_copy(xyz_hbm.at[pl.ds((b * 3 + 0) * N, N)], x_v)
            pltpu.sync_copy(xyz_hbm.at[pl.ds((b * 3 + 1) * N, N)], y_v)
            pltpu.sync_copy(xyz_hbm.at[pl.ds((b * 3 + 2) * N, N)], z_v)

            @plsc.parallel_loop(0, EW, 2 * LANES, unroll=4)
            def body(off):
                pos = off + 2 * lane
                ie = plsc.load_gather(idx_v, [pos])
                io = plsc.load_gather(idx_v, [pos + 1])
                pe = jnp.minimum(lax.shift_right_logical(pos + base, shift), N - 1)
                po = jnp.minimum(lax.shift_right_logical(pos + base + 1, shift), N - 1)

                def pair(src_v, a_idx, b_idx):
                    a = plsc.load_gather(src_v, [a_idx])
                    c = plsc.load_gather(src_v, [b_idx])
                    return plsc.pack(a, c, format=plsc.PackFormat.INTERLEAVED)

                px_v[pl.ds(off, 2 * LANES)] = pair(x_v, pe, po)
                py_v[pl.ds(off, 2 * LANES)] = pair(y_v, pe, po)
                pz_v[pl.ds(off, 2 * LANES)] = pair(z_v, pe, po)
                qx_v[pl.ds(off, 2 * LANES)] = pair(x_v, ie, io)
                qy_v[pl.ds(off, 2 * LANES)] = pair(y_v, ie, io)
                qz_v[pl.ds(off, 2 * LANES)] = pair(z_v, ie, io)
            pltpu.sync_copy(px_v, out_hbm.at[pl.ds((b * 6 + 0) * NKp + base, EW)])
            pltpu.sync_copy(py_v, out_hbm.at[pl.ds((b * 6 + 1) * NKp + base, EW)])
            pltpu.sync_copy(pz_v, out_hbm.at[pl.ds((b * 6 + 2) * NKp + base, EW)])
            pltpu.sync_copy(qx_v, out_hbm.at[pl.ds((b * 6 + 3) * NKp + base, EW)])
            pltpu.sync_copy(qy_v, out_hbm.at[pl.ds((b * 6 + 4) * NKp + base, EW)])
            pltpu.sync_copy(qz_v, out_hbm.at[pl.ds((b * 6 + 5) * NKp + base, EW)])

    return k(xyz_t.reshape(B * 3 * N), idx_pad.reshape(B * NKp)).reshape(2 * B, 3, NKp)


def _z8(p_ref, q_ref, BLK):
    P = p_ref[0].astype(jnp.float32)
    Q = q_ref[0].astype(jnp.float32)
    rel = P - Q
    s = jnp.sum(rel * rel, axis=0, keepdims=True)
    dist = s * lax.rsqrt(s + 1e-30)
    ones = jnp.ones((1, BLK), jnp.float32)
    return jnp.concatenate([P, Q, dist, ones], axis=0)


def _tc_stats(pq, W8, g2, be2, B, NK, BLK, T):
    inv_cnt = 1.0 / float(B * NK)

    def body(p_ref, q_ref, w_ref, g_ref, be_ref, wp_ref, m_ref):
        b, t = pl.program_id(0), pl.program_id(1)
        Z = _z8(p_ref, q_ref, BLK)
        m = lax.dot_general(Z, Z, (((1,), (1,)), ((), ())),
                            preferred_element_type=jnp.float32)

        @pl.when((b == 0) & (t == 0))
        def _():
            m_ref[...] = jnp.zeros_like(m_ref)

        m_ref[...] += m

        @pl.when((b == B - 1) & (t == T - 1))
        def _():
            w8 = w_ref[...]
            wm = jnp.dot(w8, m_ref[...], preferred_element_type=jnp.float32,
                         precision=lax.Precision.HIGHEST)
            mean = wm[:, 7:8] * inv_cnt
            ey2 = jnp.sum(wm * w8, axis=1, keepdims=True) * inv_cnt
            var = ey2 - mean * mean
            scale = g_ref[...] / jnp.sqrt(var + EPS_BN)
            shift = be_ref[...] - scale * mean
            col = lax.broadcasted_iota(jnp.int32, (16, 8), 1)
            wp_ref[...] = scale * w8 + jnp.where(col == 7, shift, 0.0)

    return pl.pallas_call(
        body,
        grid=(B, T),
        in_specs=[
            pl.BlockSpec((1, 3, BLK), lambda b, t: (2 * b, 0, t)),
            pl.BlockSpec((1, 3, BLK), lambda b, t: (2 * b + 1, 0, t)),
            pl.BlockSpec((16, 8), lambda b, t: (0, 0)),
            pl.BlockSpec((16, 1), lambda b, t: (0, 0)),
            pl.BlockSpec((16, 1), lambda b, t: (0, 0)),
        ],
        out_specs=pl.BlockSpec((16, 8), lambda b, t: (0, 0)),
        out_shape=jax.ShapeDtypeStruct((16, 8), jnp.float32),
        scratch_shapes=[pltpu.VMEM((8, 8), jnp.float32)],
    )(pq, pq, W8, g2, be2)


def _tc_emit(pq, W8p, B, NK, BLK, T):

    def body(p_ref, q_ref, w_ref, o_ref):
        Z = _z8(p_ref, q_ref, BLK)
        yn = jnp.dot(w_ref[...], Z, preferred_element_type=jnp.float32)
        o_ref[0] = jnp.maximum(yn, NEG_SLOPE * yn)

    return pl.pallas_call(
        body,
        grid=(B, T),
        in_specs=[
            pl.BlockSpec((1, 3, BLK), lambda b, t: (2 * b, 0, t)),
            pl.BlockSpec((1, 3, BLK), lambda b, t: (2 * b + 1, 0, t)),
            pl.BlockSpec((16, 8), lambda b, t: (0, 0)),
        ],
        out_specs=pl.BlockSpec((1, 16, BLK), lambda b, t: (b, 0, t)),
        out_shape=jax.ShapeDtypeStruct((B, 16, NK), jnp.float32),
    )(pq, pq, W8p)


def kernel(xyz_t, neighbor_idx, W, b, gamma, beta):
    B, _, N = xyz_t.shape
    K = neighbor_idx.shape[-1]
    NK = N * K
    idx_flat = neighbor_idx.reshape(B, NK).astype(jnp.int32)

    pq = _sc_gather(xyz_t, idx_flat, K)

    BLK = 64000
    assert NK % BLK == 0
    T = NK // BLK

    W8 = jnp.concatenate(
        [W[:, 1:4] + W[:, 4:7], W[:, 7:10] - W[:, 1:4], W[:, 0:1],
         b.reshape(16, 1)], axis=1)

    g2 = gamma.reshape(16, 1)
    be2 = beta.reshape(16, 1)
    W8p = _tc_stats(pq, W8, g2, be2, B, NK, BLK, T)
    return _tc_emit(pq, W8p, B, NK, BLK, T)
